# baseline (speedup 1.0000x reference)
import functools

import jax
import jax.numpy as jnp
from jax import lax
from jax.experimental import pallas as pl
from jax.experimental.pallas import tpu as pltpu
from jax.experimental.pallas import tpu_sc as plsc

_BATCH, _SEQ, _D = 16384, 20, 64
_V = 1000000
_B = _BATCH * _SEQ
_NC, _NS = 2, 16
_NW = _NC * _NS
_L = 16

_CH = 256
_NCHF = (_V - 64) // _CH
_CPT = (_NCHF + _NW - 1) // _NW

_BPW = _B // _NW
_PPW = _BPW // 2
_GP = 64
_NG = _PPW // _GP
_NBUF = 3
_BBL = _BATCH // _NW
_MT0 = (_V - 64) // 2


def _make_pack():
    mesh = plsc.VectorSubcoreMesh(core_axis_name="c", subcore_axis_name="s")

    @functools.partial(
        pl.kernel,
        mesh=mesh,
        out_type=jax.ShapeDtypeStruct((_V // 2, 128), jnp.float32),
        scratch_types=[
            pltpu.VMEM((2, _D, _CH), jnp.float32),
            pltpu.VMEM((2, _CH // 2, 128), jnp.float32),
            pltpu.SemaphoreType.DMA((2,)),
            pltpu.SemaphoreType.DMA((2,)),
        ],
        compiler_params=pltpu.CompilerParams(use_tc_tiling_on_sc=True, needs_layout_passes=False),
    )
    def pack(wt_hbm, wp_hbm, chv, outv, isem, osem):
        wid = lax.axis_index("s") * _NC + lax.axis_index("c")
        lanes = lax.iota(jnp.int32, _L)

        def start_in(t, sl):
            k = t * _NW + wid
            pltpu.make_async_copy(
                wt_hbm.at[:, pl.ds(k * _CH, _CH)], chv.at[sl], isem.at[sl]
            ).start()

        def wait_in(sl):
            pltpu.make_async_copy(
                wt_hbm.at[:, pl.ds(0, _CH)], chv.at[sl], isem.at[sl]).wait()

        def start_out(t, sl):
            k = t * _NW + wid
            pltpu.make_async_copy(
                outv.at[sl], wp_hbm.at[pl.ds(k * (_CH // 2), _CH // 2)],
                osem.at[sl]).start()

        def wait_out(sl):
            pltpu.make_async_copy(
                outv.at[sl], wp_hbm.at[pl.ds(0, _CH // 2)], osem.at[sl]).wait()

        def transpose_chunk(sl):
            def row_body(m, carry):
                def vec_body(v, c):
                    col = 2 * m + jnp.where(v >= 4, 1, 0)
                    src_r = 16 * lax.rem(v, 4) + lanes
                    val = plsc.load_gather(
                        chv, [sl + 0 * lanes, src_r, col + 0 * lanes])
                    outv[sl, m, pl.ds(v * _L, _L)] = val
                    return c
                return lax.fori_loop(0, 8, vec_body, carry)
            lax.fori_loop(0, _CH // 2, row_body, 0)

        start_in(0, 0)

        def body(t, carry):
            @pl.when(t * _NW + wid < _NCHF)
            def _():
                sl = lax.rem(t, 2)
                wait_in(sl)

                @pl.when((t + 1) * _NW + wid < _NCHF)
                def _():
                    start_in(t + 1, 1 - sl)

                @pl.when(t >= 2)
                def _():
                    wait_out(sl)

                transpose_chunk(sl)
                start_out(t, sl)
            return carry

        lax.fori_loop(0, _CPT, body, 0)
        wait_out(0)
        wait_out(1)

    return pack


def _make_gather():
    mesh = plsc.VectorSubcoreMesh(core_axis_name="c", subcore_axis_name="s")

    @functools.partial(
        pl.kernel,
        mesh=mesh,
        out_type=jax.ShapeDtypeStruct((_B // 2, 128), jnp.float32),
        scratch_types=[
            pltpu.VMEM((_NBUF * 128 + 2 * _GP + 32, 128), jnp.float32),
            pltpu.VMEM((_SEQ, _BBL), jnp.int32),
            pltpu.VMEM((_NG, 128), jnp.int32),
            pltpu.VMEM((_NG, 128), jnp.int32),
            pltpu.SemaphoreType.DMA,
            pltpu.SemaphoreType.DMA((_NBUF,)),
            pltpu.SemaphoreType.DMA((2,)),
        ],
        compiler_params=pltpu.CompilerParams(use_tc_tiling_on_sc=True, needs_layout_passes=False),
    )
    def gather(ids_hbm, wp_hbm, wtl_hbm, out_hbm, arena, idsv, ridx, hbuf,
               ssem, gsem, osem):
        wid = lax.axis_index("s") * _NC + lax.axis_index("c")
        lanes = lax.iota(jnp.int32, _L)
        ob_r = _NBUF * 128
        tail_r = _NBUF * 128 + 2 * _GP

        pltpu.make_async_copy(wtl_hbm, arena.at[pl.ds(tail_r, 32)],
                              ssem).start()
        for s in range(_SEQ):
            pltpu.make_async_copy(
                ids_hbm.at[pl.ds(s * _BATCH + wid * _BBL, _BBL)],
                idsv.at[s], ssem).start()
        pltpu.make_async_copy(wtl_hbm, arena.at[pl.ds(tail_r, 32)],
                              ssem).wait()
        for s in range(_SEQ):
            pltpu.make_async_copy(
                ids_hbm.at[pl.ds(0, _BBL)], idsv.at[s], ssem).wait()

        def pre_group(g, carry):
            def pre_vec(q, c):
                p = g * _GP + q * _L + lanes
                pd = jnp.right_shift(p * 6554, 16)
                s0 = 2 * (p - pd * 10)
                id0 = plsc.load_gather(idsv, [s0, pd])
                id1 = plsc.load_gather(idsv, [s0 + 1, pd])
                r2 = 2 * (q * _L + lanes)
                gv = g + 0 * lanes
                plsc.store_scatter(ridx, [gv, r2], jnp.right_shift(id0, 1))
                plsc.store_scatter(ridx, [gv, r2 + 1],
                                   jnp.right_shift(id1, 1))
                plsc.store_scatter(hbuf, [gv, r2], jnp.bitwise_and(id0, 1))
                plsc.store_scatter(hbuf, [gv, r2 + 1],
                                   jnp.bitwise_and(id1, 1))
                return c
            return lax.fori_loop(0, _GP // _L, pre_vec, carry)

        lax.fori_loop(0, _NG, pre_group, 0)

        def start_gather(g, sl):
            pltpu.make_async_copy(
                wp_hbm.at[ridx.at[g]], arena.at[pl.ds(sl * 128, 128)],
                gsem.at[sl]).start()

        def wait_gather(sl):
            pltpu.make_async_copy(
                wp_hbm.at[pl.ds(0, 128)], arena.at[pl.ds(sl * 128, 128)],
                gsem.at[sl]).wait()

        def start_out(g, sl):
            pltpu.make_async_copy(
                arena.at[pl.ds(ob_r + sl * _GP, _GP)],
                out_hbm.at[pl.ds(wid * _PPW + g * _GP, _GP)],
                osem.at[sl]).start()

        def wait_out(sl):
            pltpu.make_async_copy(
                arena.at[pl.ds(ob_r, _GP)], out_hbm.at[pl.ds(0, _GP)],
                osem.at[sl]).wait()

        def assemble(g, gsl, osl):
            def vec_q(q, carry):
                r2 = 2 * (q * _L + lanes)
                orow = ob_r + osl * _GP + q * _L + lanes
                gv = g + 0 * lanes

                def do_half(off):
                    rloc = r2 + off
                    rr = plsc.load_gather(ridx, [gv, rloc])
                    hh = plsc.load_gather(hbuf, [gv, rloc])
                    base_row = jnp.where(rr >= _MT0, tail_r + (rr - _MT0),
                                         gsl * 128 + rloc)
                    col0 = hh * _D

                    def dloop(dv, c2):
                        val = plsc.load_gather(arena, [base_row, col0 + dv])
                        plsc.store_scatter(
                            arena, [orow, off * _D + dv + 0 * lanes], val)
                        return c2
                    lax.fori_loop(0, _D, dloop, 0)

                do_half(0)
                do_half(1)
                return carry
            return lax.fori_loop(0, _GP // _L, vec_q, 0)

        for sl in range(_NBUF - 1):
            start_gather(sl, sl)

        def body(g, carry):
            gsl = lax.rem(g, _NBUF)

            @pl.when(g + _NBUF - 1 < _NG)
            def _():
                start_gather(g + _NBUF - 1, lax.rem(g + _NBUF - 1, _NBUF))

            wait_gather(gsl)
            osl = lax.rem(g, 2)

            @pl.when(g >= 2)
            def _():
                wait_out(osl)

            assemble(g, gsl, osl)
            start_out(g, osl)
            return carry

        lax.fori_loop(0, _NG, body, 0)
        wait_out(0)
        wait_out(1)

    return gather


_pack = _make_pack()
_gather = _make_gather()


def kernel(input_ids, W):
    ids_f = jnp.transpose(input_ids).astype(jnp.int32).reshape(-1)
    wt = jnp.transpose(W)
    wtl = W[_V - 64:].reshape(32, 128)
    wp = _pack(wt)
    out = _gather(ids_f, wp, wtl)
    return out.reshape(_BATCH, _SEQ, _D)


# speedup vs baseline: 2.9417x; 2.9417x over previous
import functools

import jax
import jax.numpy as jnp
from jax import lax
from jax.experimental import pallas as pl
from jax.experimental.pallas import tpu as pltpu
from jax.experimental.pallas import tpu_sc as plsc

_BATCH, _SEQ, _D = 16384, 20, 64
_B = _BATCH * _SEQ
_NC, _NS = 2, 16
_NW = _NC * _NS
_BPW = _B // _NW
_G = 128
_NG = _BPW // _G
_NBUF = 8
_K = 4
_NT = _NG // _NBUF
_L = 16


def _make_lookup():
    mesh = plsc.VectorSubcoreMesh(core_axis_name="c", subcore_axis_name="s")

    @functools.partial(
        pl.kernel,
        mesh=mesh,
        out_type=jax.ShapeDtypeStruct((_B, _D), jnp.float32),
        scratch_types=[
            pltpu.VMEM((_NG, _G), jnp.int32),
            pltpu.VMEM((_NG, _G), jnp.int32),
            pltpu.VMEM((_NBUF, _G, _D), jnp.float32),
            pltpu.SemaphoreType.DMA((_NBUF,)),
            pltpu.SemaphoreType.DMA((_NBUF,)),
        ],
        compiler_params=pltpu.CompilerParams(use_tc_tiling_on_sc=False),
    )
    def lookup(ids_hbm, table_hbm, out_hbm, idx_v, jdx_v, rows_v, gsem, osem):
        wid = lax.axis_index("s") * _NC + lax.axis_index("c")
        base = wid * _BPW
        pltpu.sync_copy(ids_hbm.at[wid], idx_v)

        def fill_jdx_loop(g, carry):
            def inner(u, c):
                jt = base + g * _G + u * _L + lax.iota(jnp.int32, _L)
                b = jnp.bitwise_and(jt, _BATCH - 1)
                s = jnp.right_shift(jt, 14)
                jdx_v[g, pl.ds(u * _L, _L)] = b * _SEQ + s
                return c
            return lax.fori_loop(0, _G // _L, inner, carry)

        lax.fori_loop(0, _NG, fill_jdx_loop, 0)

        def start_gather(g, b):
            pltpu.make_async_copy(
                table_hbm.at[idx_v.at[g]], rows_v.at[b], gsem.at[b]).start()

        def wait_gather(b):
            pltpu.make_async_copy(
                table_hbm.at[pl.ds(0, _G)], rows_v.at[b], gsem.at[b]).wait()

        def start_out(g, b):
            pltpu.make_async_copy(
                rows_v.at[b], out_hbm.at[jdx_v.at[g]], osem.at[b]).start()

        def wait_out(b):
            pltpu.make_async_copy(
                rows_v.at[b], out_hbm.at[pl.ds(0, _G)], osem.at[b]).wait()

        for b in range(_K):
            start_gather(b, b)

        for b in range(_NBUF):
            wait_gather(b)
            start_out(b, b)
            s4 = (b + _K) % _NBUF
            if b < _K:
                start_gather(b + _K, s4)
            else:
                wait_out(s4)
                start_gather(b + _K, s4)

        def revolution(t, carry):
            for b in range(_NBUF):
                g = t * _NBUF + b
                wait_gather(b)
                start_out(g, b)
                s4 = (b + _K) % _NBUF
                wait_out(s4)
                start_gather(g + _K, s4)
            return carry

        lax.fori_loop(1, _NT - 1, revolution, 0)

        for b in range(_NBUF):
            g = (_NT - 1) * _NBUF + b
            wait_gather(b)
            start_out(g, b)
            if b < _K:
                s4 = (b + _K) % _NBUF
                wait_out(s4)
                start_gather(g + _K, s4)

        for b in range(_NBUF):
            wait_out(b)

    return lookup


_lookup = _make_lookup()


def kernel(input_ids, W):
    ids_t = jnp.transpose(input_ids).astype(jnp.int32)
    ids = ids_t.reshape(_NW, _NG, _G)
    out = _lookup(ids, W)
    return out.reshape(_BATCH, _SEQ, _D)
